# baseline (speedup 1.0000x reference)
import functools

import jax
import jax.numpy as jnp
from jax import lax
from jax.experimental import pallas as pl
from jax.experimental.pallas import tpu as pltpu
from jax.experimental.pallas import tpu_sc as plsc

N = 10000
E = 160000
D = 256
G = 128
DH = D // 2

NC, NS = 2, 16
NW = NC * NS

KE = 128
E_PAD = 163840
NPAD = 10240
ROWS_PT = NPAD // NS
DEG_PT = 640
DEG_PAD = DEG_PT * NS

KA = E_PAD // NW


def _deg_body(col_hbm, zeros_hbm, ones_hbm, out_hbm, colbuf, onesbuf, acc_sh, sem):
    c = lax.axis_index("c")
    s = lax.axis_index("s")
    wid = c * NS + s
    pltpu.sync_copy(zeros_hbm, acc_sh.at[pl.ds(s * DEG_PT, DEG_PT)])
    pltpu.sync_copy(ones_hbm, onesbuf)
    pltpu.sync_copy(col_hbm.at[pl.ds(wid * KA, KA)], colbuf)
    plsc.subcore_barrier()
    pltpu.sync_copy(onesbuf, acc_sh.at[colbuf], add=True)
    plsc.subcore_barrier()

    @pl.when(c == 0)
    def _():
        pltpu.sync_copy(acc_sh.at[pl.ds(s * DEG_PT, DEG_PT)],
                        out_hbm.at[0, pl.ds(s * DEG_PT, DEG_PT)])

    @pl.when(c == 1)
    def _():
        pltpu.sync_copy(acc_sh.at[pl.ds(s * DEG_PT, DEG_PT)],
                        out_hbm.at[1, pl.ds(s * DEG_PT, DEG_PT)])


@functools.cache
def _deg_call():
    mesh = plsc.VectorSubcoreMesh(core_axis_name="c", subcore_axis_name="s",
                                  num_cores=NC, num_subcores=NS)
    return pl.kernel(
        _deg_body,
        out_type=jax.ShapeDtypeStruct((2, DEG_PAD), jnp.float32),
        mesh=mesh,
        scratch_types=[
            pltpu.VMEM((KA,), jnp.int32),
            pltpu.VMEM((KA,), jnp.float32),
            pltpu.VMEM_SHARED((DEG_PAD,), jnp.float32),
            pltpu.SemaphoreType.DMA,
        ],
    )


NCH = E_PAD // NS // KE
PH = 2
NCHP = NCH // PH


def _scatter_body(row_hbm, col_hbm, y0_hbm, y1_hbm, zrows_hbm, z0_hbm, z1_hbm,
                  rowsb, colsb, gbuf0, gbuf1, acc_sh, sem0, sem1):
    c = lax.axis_index("c")
    s = lax.axis_index("s")
    ept = E_PAD // NS
    base = s * ept
    pltpu.sync_copy(zrows_hbm, acc_sh.at[pl.ds(s * ROWS_PT, ROWS_PT), :])
    plsc.subcore_barrier()

    def run(y_hbm):
        def phase(ph, carry0):
            pbase = pl.multiple_of(base + ph * NCHP * KE, 8)
            pltpu.sync_copy(row_hbm.at[pl.ds(pbase, NCHP * KE)], rowsb)
            pltpu.sync_copy(col_hbm.at[pl.ds(pbase, NCHP * KE)], colsb)
            pltpu.async_copy(y_hbm.at[rowsb.at[pl.ds(0, KE)]], gbuf0, sem0)

            def pair(j, carry):
                o0 = pl.multiple_of(2 * j * KE, 128)
                o1 = pl.multiple_of((2 * j + 1) * KE, 128)
                o2 = pl.multiple_of((2 * j + 2) * KE, 128)
                pltpu.async_copy(y_hbm.at[rowsb.at[pl.ds(o1, KE)]], gbuf1, sem1)
                pltpu.make_async_copy(y_hbm.at[rowsb.at[pl.ds(o0, KE)]],
                                      gbuf0, sem0).wait()
                pltpu.sync_copy(gbuf0, acc_sh.at[colsb.at[pl.ds(o0, KE)]],
                                add=True)

                @pl.when(j < NCHP // 2 - 1)
                def _():
                    pltpu.async_copy(y_hbm.at[rowsb.at[pl.ds(o2, KE)]],
                                     gbuf0, sem0)

                pltpu.make_async_copy(y_hbm.at[rowsb.at[pl.ds(o1, KE)]],
                                      gbuf1, sem1).wait()
                pltpu.sync_copy(gbuf1, acc_sh.at[colsb.at[pl.ds(o1, KE)]],
                                add=True)
                return carry

            lax.fori_loop(0, NCHP // 2, pair, 0)
            return carry0

        lax.fori_loop(0, PH, phase, 0)

    @pl.when(c == 0)
    def _():
        run(y0_hbm)

    @pl.when(c == 1)
    def _():
        run(y1_hbm)

    plsc.subcore_barrier()

    @pl.when(c == 0)
    def _():
        pltpu.sync_copy(acc_sh.at[pl.ds(s * ROWS_PT, ROWS_PT), :],
                        z0_hbm.at[pl.ds(s * ROWS_PT, ROWS_PT), :])

    @pl.when(c == 1)
    def _():
        pltpu.sync_copy(acc_sh.at[pl.ds(s * ROWS_PT, ROWS_PT), :],
                        z1_hbm.at[pl.ds(s * ROWS_PT, ROWS_PT), :])


@functools.cache
def _scatter_call():
    mesh = plsc.VectorSubcoreMesh(core_axis_name="c", subcore_axis_name="s",
                                  num_cores=NC, num_subcores=NS)
    return pl.kernel(
        _scatter_body,
        out_type=[jax.ShapeDtypeStruct((NPAD, DH), jnp.float32),
                  jax.ShapeDtypeStruct((NPAD, DH), jnp.float32)],
        mesh=mesh,
        scratch_types=[
            pltpu.VMEM((NCHP * KE,), jnp.int32),
            pltpu.VMEM((NCHP * KE,), jnp.int32),
            pltpu.VMEM((KE, DH), jnp.float32),
            pltpu.VMEM((KE, DH), jnp.float32),
            pltpu.VMEM_SHARED((NPAD, DH), jnp.float32),
            pltpu.SemaphoreType.DMA,
            pltpu.SemaphoreType.DMA,
        ],
    )


BV = 1000


def _mm_body(d0_ref, d1_ref, v_ref, w_ref, y0_ref, y1_ref):
    dis = lax.rsqrt(d0_ref[...] + d1_ref[...] + 1.0)
    x = jnp.dot(v_ref[...].astype(jnp.bfloat16), w_ref[...].astype(jnp.bfloat16),
                preferred_element_type=jnp.float32)
    y = x * dis
    y0_ref[...] = y[:, :DH]
    y1_ref[...] = y[:, DH:]


def _mm_call(d0, d1, v, W):
    return pl.pallas_call(
        _mm_body,
        grid=(N // BV,),
        in_specs=[
            pl.BlockSpec((BV, 1), lambda i: (i, 0)),
            pl.BlockSpec((BV, 1), lambda i: (i, 0)),
            pl.BlockSpec((BV, D), lambda i: (i, 0)),
            pl.BlockSpec((D, D), lambda i: (0, 0)),
        ],
        out_specs=[
            pl.BlockSpec((BV, DH), lambda i: (i, 0)),
            pl.BlockSpec((BV, DH), lambda i: (i, 0)),
        ],
        out_shape=[jax.ShapeDtypeStruct((N, DH), jnp.float32),
                   jax.ShapeDtypeStruct((N, DH), jnp.float32)],
    )(d0, d1, v, W)


def _pool_body(batch_ref, d0_ref, d1_ref, z0_ref, z1_ref, y0_ref, y1_ref,
               b_ref, out_ref, cnt_ref):
    i = pl.program_id(0)

    @pl.when(i == 0)
    def _():
        out_ref[...] = jnp.zeros_like(out_ref)
        cnt_ref[...] = jnp.zeros_like(cnt_ref)

    dis = lax.rsqrt(d0_ref[...] + d1_ref[...] + 1.0)
    h0 = jnp.maximum(dis * (z0_ref[...] + y0_ref[...]) + b_ref[:, :DH], 0.0)
    h1 = jnp.maximum(dis * (z1_ref[...] + y1_ref[...]) + b_ref[:, DH:], 0.0)
    bt = batch_ref[0]
    pt = (lax.broadcasted_iota(jnp.int32, (G, BV), 0) == bt).astype(jnp.bfloat16)
    out_ref[:, :DH] += jnp.dot(pt, h0.astype(jnp.bfloat16),
                               preferred_element_type=jnp.float32)
    out_ref[:, DH:] += jnp.dot(pt, h1.astype(jnp.bfloat16),
                               preferred_element_type=jnp.float32)
    cnt_ref[...] += jnp.sum(pt.astype(jnp.float32), axis=1, keepdims=True)

    @pl.when(i == pl.num_programs(0) - 1)
    def _():
        out_ref[...] = out_ref[...] / jnp.maximum(cnt_ref[...], 1.0)


def _pool_call(batch3, d0, d1, z0, z1, y0, y1, b2):
    return pl.pallas_call(
        _pool_body,
        grid=(N // BV,),
        in_specs=[
            pl.BlockSpec((1, 1, BV), lambda i: (i, 0, 0)),
            pl.BlockSpec((BV, 1), lambda i: (i, 0)),
            pl.BlockSpec((BV, 1), lambda i: (i, 0)),
            pl.BlockSpec((BV, DH), lambda i: (i, 0)),
            pl.BlockSpec((BV, DH), lambda i: (i, 0)),
            pl.BlockSpec((BV, DH), lambda i: (i, 0)),
            pl.BlockSpec((BV, DH), lambda i: (i, 0)),
            pl.BlockSpec((1, D), lambda i: (0, 0)),
        ],
        out_specs=pl.BlockSpec((G, D), lambda i: (0, 0)),
        out_shape=jax.ShapeDtypeStruct((G, D), jnp.float32),
        scratch_shapes=[pltpu.VMEM((G, 1), jnp.float32)],
    )(batch3, d0, d1, z0, z1, y0, y1, b2)


def kernel(v, e, batch, W, b):
    e = e.astype(jnp.int32)
    row, col = e[0], e[1]
    npd = E_PAD - E
    prow = jnp.arange(npd, dtype=jnp.int32) % N
    pcol = N + (jnp.arange(npd, dtype=jnp.int32) % (NPAD - N))
    rowp = jnp.concatenate([row, prow])
    colp = jnp.concatenate([col, pcol])
    zeros_deg = jnp.zeros((DEG_PT,), jnp.float32)
    ones_ka = jnp.ones((KA,), jnp.float32)
    zeros_rows = jnp.zeros((ROWS_PT, DH), jnp.float32)

    degp = _deg_call()(colp, zeros_deg, ones_ka)
    d0 = degp[0].reshape(DEG_PAD, 1)
    d1 = degp[1].reshape(DEG_PAD, 1)

    y0, y1 = _mm_call(d0, d1, v, W)
    z0, z1 = _scatter_call()(rowp, colp, y0, y1, zeros_rows)

    batch3 = batch.astype(jnp.int32).reshape(N // BV, 1, BV)
    b2 = b.reshape(1, D)
    return _pool_call(batch3, d0, d1, z0, z1, y0, y1, b2)


# speedup vs baseline: 1.0171x; 1.0171x over previous
import functools

import jax
import jax.numpy as jnp
from jax import lax
from jax.experimental import pallas as pl
from jax.experimental.pallas import tpu as pltpu
from jax.experimental.pallas import tpu_sc as plsc

N = 10000
E = 160000
D = 256
G = 128
DH = D // 2

NC, NS = 2, 16
NW = NC * NS

KE = 128
E_PAD = 163840
NPAD = 10240
ROWS_PT = NPAD // NS
DEG_PT = 640
DEG_PAD = DEG_PT * NS

KA = E_PAD // NW


def _deg_body(col_hbm, zeros_hbm, ones_hbm, out_hbm, colbuf, onesbuf, acc_sh, sem):
    c = lax.axis_index("c")
    s = lax.axis_index("s")
    wid = c * NS + s
    pltpu.sync_copy(zeros_hbm, acc_sh.at[pl.ds(s * DEG_PT, DEG_PT)])
    pltpu.sync_copy(ones_hbm, onesbuf)
    pltpu.sync_copy(col_hbm.at[pl.ds(wid * KA, KA)], colbuf)
    plsc.subcore_barrier()
    pltpu.sync_copy(onesbuf, acc_sh.at[colbuf], add=True)
    plsc.subcore_barrier()

    @pl.when(c == 0)
    def _():
        pltpu.sync_copy(acc_sh.at[pl.ds(s * DEG_PT, DEG_PT)],
                        out_hbm.at[0, pl.ds(s * DEG_PT, DEG_PT)])

    @pl.when(c == 1)
    def _():
        pltpu.sync_copy(acc_sh.at[pl.ds(s * DEG_PT, DEG_PT)],
                        out_hbm.at[1, pl.ds(s * DEG_PT, DEG_PT)])


@functools.cache
def _deg_call():
    mesh = plsc.VectorSubcoreMesh(core_axis_name="c", subcore_axis_name="s",
                                  num_cores=NC, num_subcores=NS)
    return pl.kernel(
        _deg_body,
        out_type=jax.ShapeDtypeStruct((2, DEG_PAD), jnp.float32),
        mesh=mesh,
        scratch_types=[
            pltpu.VMEM((KA,), jnp.int32),
            pltpu.VMEM((KA,), jnp.float32),
            pltpu.VMEM_SHARED((DEG_PAD,), jnp.float32),
            pltpu.SemaphoreType.DMA,
        ],
    )


NCH = E_PAD // NS // KE
PH = 2
NCHP = NCH // PH


def _scatter_body(row_hbm, col_hbm, y0_hbm, y1_hbm, zrows_hbm, z0_hbm, z1_hbm,
                  rowsb, colsb, gbuf0, gbuf1, acc_sh, sem0, sem1):
    c = lax.axis_index("c")
    s = lax.axis_index("s")
    ept = E_PAD // NS
    base = s * ept
    pltpu.sync_copy(zrows_hbm, acc_sh.at[pl.ds(s * ROWS_PT, ROWS_PT), :])
    plsc.subcore_barrier()

    def run(y_hbm):
        def phase(ph, carry0):
            pbase = pl.multiple_of(base + ph * NCHP * KE, 8)
            pltpu.sync_copy(row_hbm.at[pl.ds(pbase, NCHP * KE)], rowsb)
            pltpu.sync_copy(col_hbm.at[pl.ds(pbase, NCHP * KE)], colsb)
            pltpu.async_copy(y_hbm.at[rowsb.at[pl.ds(0, KE)]], gbuf0, sem0)

            def pair(j, carry):
                o0 = pl.multiple_of(2 * j * KE, 128)
                o1 = pl.multiple_of((2 * j + 1) * KE, 128)
                o2 = pl.multiple_of((2 * j + 2) * KE, 128)
                pltpu.async_copy(y_hbm.at[rowsb.at[pl.ds(o1, KE)]], gbuf1, sem1)
                pltpu.make_async_copy(y_hbm.at[rowsb.at[pl.ds(o0, KE)]],
                                      gbuf0, sem0).wait()
                pltpu.sync_copy(gbuf0, acc_sh.at[colsb.at[pl.ds(o0, KE)]],
                                add=True)

                @pl.when(j < NCHP // 2 - 1)
                def _():
                    pltpu.async_copy(y_hbm.at[rowsb.at[pl.ds(o2, KE)]],
                                     gbuf0, sem0)

                pltpu.make_async_copy(y_hbm.at[rowsb.at[pl.ds(o1, KE)]],
                                      gbuf1, sem1).wait()
                pltpu.sync_copy(gbuf1, acc_sh.at[colsb.at[pl.ds(o1, KE)]],
                                add=True)
                return carry

            lax.fori_loop(0, NCHP // 2, pair, 0)
            return carry0

        lax.fori_loop(0, PH, phase, 0)

    @pl.when(c == 0)
    def _():
        run(y0_hbm)

    @pl.when(c == 1)
    def _():
        run(y1_hbm)

    plsc.subcore_barrier()

    @pl.when(c == 0)
    def _():
        pltpu.sync_copy(acc_sh.at[pl.ds(s * ROWS_PT, ROWS_PT), :],
                        z0_hbm.at[pl.ds(s * ROWS_PT, ROWS_PT), :])

    @pl.when(c == 1)
    def _():
        pltpu.sync_copy(acc_sh.at[pl.ds(s * ROWS_PT, ROWS_PT), :],
                        z1_hbm.at[pl.ds(s * ROWS_PT, ROWS_PT), :])


@functools.cache
def _scatter_call():
    mesh = plsc.VectorSubcoreMesh(core_axis_name="c", subcore_axis_name="s",
                                  num_cores=NC, num_subcores=NS)
    return pl.kernel(
        _scatter_body,
        out_type=[jax.ShapeDtypeStruct((NPAD, DH), jnp.float32),
                  jax.ShapeDtypeStruct((NPAD, DH), jnp.float32)],
        mesh=mesh,
        scratch_types=[
            pltpu.VMEM((NCHP * KE,), jnp.int32),
            pltpu.VMEM((NCHP * KE,), jnp.int32),
            pltpu.VMEM((KE, DH), jnp.float32),
            pltpu.VMEM((KE, DH), jnp.float32),
            pltpu.VMEM_SHARED((NPAD, DH), jnp.float32),
            pltpu.SemaphoreType.DMA,
            pltpu.SemaphoreType.DMA,
        ],
    )


BV = 2000


def _mm_body(d0_ref, d1_ref, v_ref, w_ref, y0_ref, y1_ref):
    dis = lax.rsqrt(d0_ref[...] + d1_ref[...] + 1.0)
    x = jnp.dot(v_ref[...].astype(jnp.bfloat16), w_ref[...].astype(jnp.bfloat16),
                preferred_element_type=jnp.float32)
    y = x * dis
    y0_ref[...] = y[:, :DH]
    y1_ref[...] = y[:, DH:]


def _mm_call(d0, d1, v, W):
    return pl.pallas_call(
        _mm_body,
        grid=(N // BV,),
        in_specs=[
            pl.BlockSpec((BV, 1), lambda i: (i, 0)),
            pl.BlockSpec((BV, 1), lambda i: (i, 0)),
            pl.BlockSpec((BV, D), lambda i: (i, 0)),
            pl.BlockSpec((D, D), lambda i: (0, 0)),
        ],
        out_specs=[
            pl.BlockSpec((BV, DH), lambda i: (i, 0)),
            pl.BlockSpec((BV, DH), lambda i: (i, 0)),
        ],
        out_shape=[jax.ShapeDtypeStruct((N, DH), jnp.float32),
                   jax.ShapeDtypeStruct((N, DH), jnp.float32)],
    )(d0, d1, v, W)


def _pool_body(batch_ref, d0_ref, d1_ref, z0_ref, z1_ref, y0_ref, y1_ref,
               b_ref, out_ref, cnt_ref):
    i = pl.program_id(0)

    @pl.when(i == 0)
    def _():
        out_ref[...] = jnp.zeros_like(out_ref)
        cnt_ref[...] = jnp.zeros_like(cnt_ref)

    dis = lax.rsqrt(d0_ref[...] + d1_ref[...] + 1.0)
    h0 = jnp.maximum(dis * (z0_ref[...] + y0_ref[...]) + b_ref[:, :DH], 0.0)
    h1 = jnp.maximum(dis * (z1_ref[...] + y1_ref[...]) + b_ref[:, DH:], 0.0)
    bt = batch_ref[0]
    pt = (lax.broadcasted_iota(jnp.int32, (G, BV), 0) == bt).astype(jnp.bfloat16)
    out_ref[:, :DH] += jnp.dot(pt, h0.astype(jnp.bfloat16),
                               preferred_element_type=jnp.float32)
    out_ref[:, DH:] += jnp.dot(pt, h1.astype(jnp.bfloat16),
                               preferred_element_type=jnp.float32)
    cnt_ref[...] += jnp.sum(pt.astype(jnp.float32), axis=1, keepdims=True)

    @pl.when(i == pl.num_programs(0) - 1)
    def _():
        out_ref[...] = out_ref[...] / jnp.maximum(cnt_ref[...], 1.0)


def _pool_call(batch3, d0, d1, z0, z1, y0, y1, b2):
    return pl.pallas_call(
        _pool_body,
        grid=(N // BV,),
        in_specs=[
            pl.BlockSpec((1, 1, BV), lambda i: (i, 0, 0)),
            pl.BlockSpec((BV, 1), lambda i: (i, 0)),
            pl.BlockSpec((BV, 1), lambda i: (i, 0)),
            pl.BlockSpec((BV, DH), lambda i: (i, 0)),
            pl.BlockSpec((BV, DH), lambda i: (i, 0)),
            pl.BlockSpec((BV, DH), lambda i: (i, 0)),
            pl.BlockSpec((BV, DH), lambda i: (i, 0)),
            pl.BlockSpec((1, D), lambda i: (0, 0)),
        ],
        out_specs=pl.BlockSpec((G, D), lambda i: (0, 0)),
        out_shape=jax.ShapeDtypeStruct((G, D), jnp.float32),
        scratch_shapes=[pltpu.VMEM((G, 1), jnp.float32)],
    )(batch3, d0, d1, z0, z1, y0, y1, b2)


def kernel(v, e, batch, W, b):
    e = e.astype(jnp.int32)
    row, col = e[0], e[1]
    npd = E_PAD - E
    prow = jnp.arange(npd, dtype=jnp.int32) % N
    pcol = N + (jnp.arange(npd, dtype=jnp.int32) % (NPAD - N))
    rowp = jnp.concatenate([row, prow])
    colp = jnp.concatenate([col, pcol])
    zeros_deg = jnp.zeros((DEG_PT,), jnp.float32)
    ones_ka = jnp.ones((KA,), jnp.float32)
    zeros_rows = jnp.zeros((ROWS_PT, DH), jnp.float32)

    degp = _deg_call()(colp, zeros_deg, ones_ka)
    d0 = degp[0].reshape(DEG_PAD, 1)
    d1 = degp[1].reshape(DEG_PAD, 1)

    y0, y1 = _mm_call(d0, d1, v, W)
    z0, z1 = _scatter_call()(rowp, colp, y0, y1, zeros_rows)

    batch3 = batch.astype(jnp.int32).reshape(N // BV, 1, BV)
    b2 = b.reshape(1, D)
    return _pool_call(batch3, d0, d1, z0, z1, y0, y1, b2)


# speedup vs baseline: 1.0332x; 1.0158x over previous
import functools

import jax
import jax.numpy as jnp
from jax import lax
from jax.experimental import pallas as pl
from jax.experimental.pallas import tpu as pltpu
from jax.experimental.pallas import tpu_sc as plsc

N = 10000
E = 160000
D = 256
G = 128
DH = D // 2

NC, NS = 2, 16
NW = NC * NS

KE = 128
E_PAD = 163840
NPAD = 10240
ROWS_PT = NPAD // NS
DEG_PT = 640
DEG_PAD = DEG_PT * NS

KA = E_PAD // NW


def _deg_body(col_hbm, zeros_hbm, ones_hbm, out_hbm, colbuf, onesbuf, acc_sh, sem):
    c = lax.axis_index("c")
    s = lax.axis_index("s")
    wid = c * NS + s
    pltpu.sync_copy(zeros_hbm, acc_sh.at[pl.ds(s * DEG_PT, DEG_PT)])
    pltpu.sync_copy(ones_hbm, onesbuf)
    pltpu.sync_copy(col_hbm.at[pl.ds(wid * KA, KA)], colbuf)
    plsc.subcore_barrier()
    pltpu.sync_copy(onesbuf, acc_sh.at[colbuf], add=True)
    plsc.subcore_barrier()

    @pl.when(c == 0)
    def _():
        pltpu.sync_copy(acc_sh.at[pl.ds(s * DEG_PT, DEG_PT)],
                        out_hbm.at[0, pl.ds(s * DEG_PT, DEG_PT)])

    @pl.when(c == 1)
    def _():
        pltpu.sync_copy(acc_sh.at[pl.ds(s * DEG_PT, DEG_PT)],
                        out_hbm.at[1, pl.ds(s * DEG_PT, DEG_PT)])


@functools.cache
def _deg_call():
    mesh = plsc.VectorSubcoreMesh(core_axis_name="c", subcore_axis_name="s",
                                  num_cores=NC, num_subcores=NS)
    return pl.kernel(
        _deg_body,
        out_type=jax.ShapeDtypeStruct((2, DEG_PAD), jnp.float32),
        mesh=mesh,
        scratch_types=[
            pltpu.VMEM((KA,), jnp.int32),
            pltpu.VMEM((KA,), jnp.float32),
            pltpu.VMEM_SHARED((DEG_PAD,), jnp.float32),
            pltpu.SemaphoreType.DMA,
        ],
    )


NCH = E_PAD // NS // KE
PH = 2
NCHP = NCH // PH


def _scatter_body(row_hbm, col_hbm, y0_hbm, y1_hbm, zrows_hbm, z0_hbm, z1_hbm,
                  rowsb, colsb, gbuf0, gbuf1, acc_sh, sem0, sem1):
    c = lax.axis_index("c")
    s = lax.axis_index("s")
    ept = E_PAD // NS
    base = s * ept
    pltpu.sync_copy(zrows_hbm, acc_sh.at[pl.ds(s * ROWS_PT, ROWS_PT), :])
    plsc.subcore_barrier()

    def run(y_hbm):
        def phase(ph, carry0):
            pbase = pl.multiple_of(base + ph * NCHP * KE, 8)
            pltpu.sync_copy(row_hbm.at[pl.ds(pbase, NCHP * KE)], rowsb)
            pltpu.sync_copy(col_hbm.at[pl.ds(pbase, NCHP * KE)], colsb)
            pltpu.async_copy(y_hbm.at[rowsb.at[pl.ds(0, KE)]], gbuf0, sem0)

            def pair(j, carry):
                o0 = pl.multiple_of(2 * j * KE, 128)
                o1 = pl.multiple_of((2 * j + 1) * KE, 128)
                o2 = pl.multiple_of((2 * j + 2) * KE, 128)
                pltpu.async_copy(y_hbm.at[rowsb.at[pl.ds(o1, KE)]], gbuf1, sem1)
                pltpu.make_async_copy(y_hbm.at[rowsb.at[pl.ds(o0, KE)]],
                                      gbuf0, sem0).wait()
                pltpu.sync_copy(gbuf0, acc_sh.at[colsb.at[pl.ds(o0, KE)]],
                                add=True)

                @pl.when(j < NCHP // 2 - 1)
                def _():
                    pltpu.async_copy(y_hbm.at[rowsb.at[pl.ds(o2, KE)]],
                                     gbuf0, sem0)

                pltpu.make_async_copy(y_hbm.at[rowsb.at[pl.ds(o1, KE)]],
                                      gbuf1, sem1).wait()
                pltpu.sync_copy(gbuf1, acc_sh.at[colsb.at[pl.ds(o1, KE)]],
                                add=True)
                return carry

            lax.fori_loop(0, NCHP // 2, pair, 0)
            return carry0

        lax.fori_loop(0, PH, phase, 0)

    @pl.when(c == 0)
    def _():
        run(y0_hbm)

    @pl.when(c == 1)
    def _():
        run(y1_hbm)

    plsc.subcore_barrier()

    @pl.when(c == 0)
    def _():
        pltpu.sync_copy(acc_sh.at[pl.ds(s * ROWS_PT, ROWS_PT), :],
                        z0_hbm.at[pl.ds(s * ROWS_PT, ROWS_PT), :])

    @pl.when(c == 1)
    def _():
        pltpu.sync_copy(acc_sh.at[pl.ds(s * ROWS_PT, ROWS_PT), :],
                        z1_hbm.at[pl.ds(s * ROWS_PT, ROWS_PT), :])


@functools.cache
def _scatter_call():
    mesh = plsc.VectorSubcoreMesh(core_axis_name="c", subcore_axis_name="s",
                                  num_cores=NC, num_subcores=NS)
    return pl.kernel(
        _scatter_body,
        out_type=[jax.ShapeDtypeStruct((NPAD, DH), jnp.float32),
                  jax.ShapeDtypeStruct((NPAD, DH), jnp.float32)],
        mesh=mesh,
        scratch_types=[
            pltpu.VMEM((NCHP * KE,), jnp.int32),
            pltpu.VMEM((NCHP * KE,), jnp.int32),
            pltpu.VMEM((KE, DH), jnp.float32),
            pltpu.VMEM((KE, DH), jnp.float32),
            pltpu.VMEM_SHARED((NPAD, DH), jnp.float32),
            pltpu.SemaphoreType.DMA,
            pltpu.SemaphoreType.DMA,
        ],
    )


BV = 2000


def _mm_body(dd_ref, v_ref, w_ref, y0_ref, y1_ref):
    dis = lax.rsqrt(dd_ref[:, 0:1] + dd_ref[:, 1:2] + 1.0)
    x = jnp.dot(v_ref[...].astype(jnp.bfloat16), w_ref[...].astype(jnp.bfloat16),
                preferred_element_type=jnp.float32)
    y = x * dis
    y0_ref[...] = y[:, :DH]
    y1_ref[...] = y[:, DH:]


def _mm_call(dd, v, W):
    return pl.pallas_call(
        _mm_body,
        grid=(N // BV,),
        in_specs=[
            pl.BlockSpec((BV, 2), lambda i: (i, 0)),
            pl.BlockSpec((BV, D), lambda i: (i, 0)),
            pl.BlockSpec((D, D), lambda i: (0, 0)),
        ],
        out_specs=[
            pl.BlockSpec((BV, DH), lambda i: (i, 0)),
            pl.BlockSpec((BV, DH), lambda i: (i, 0)),
        ],
        out_shape=[jax.ShapeDtypeStruct((N, DH), jnp.float32),
                   jax.ShapeDtypeStruct((N, DH), jnp.float32)],
    )(dd, v, W)


def _pool_body(batch_ref, dd_ref, z0_ref, z1_ref, y0_ref, y1_ref,
               b_ref, out_ref, cnt_ref):
    i = pl.program_id(0)

    @pl.when(i == 0)
    def _():
        out_ref[...] = jnp.zeros_like(out_ref)
        cnt_ref[...] = jnp.zeros_like(cnt_ref)

    dis = lax.rsqrt(dd_ref[:, 0:1] + dd_ref[:, 1:2] + 1.0)
    h0 = jnp.maximum(dis * (z0_ref[...] + y0_ref[...]) + b_ref[:, :DH], 0.0)
    h1 = jnp.maximum(dis * (z1_ref[...] + y1_ref[...]) + b_ref[:, DH:], 0.0)
    bt = batch_ref[0]
    pt = (lax.broadcasted_iota(jnp.int32, (G, BV), 0) == bt).astype(jnp.bfloat16)
    out_ref[:, :DH] += jnp.dot(pt, h0.astype(jnp.bfloat16),
                               preferred_element_type=jnp.float32)
    out_ref[:, DH:] += jnp.dot(pt, h1.astype(jnp.bfloat16),
                               preferred_element_type=jnp.float32)
    cnt_ref[...] += jnp.sum(pt.astype(jnp.float32), axis=1, keepdims=True)

    @pl.when(i == pl.num_programs(0) - 1)
    def _():
        out_ref[...] = out_ref[...] / jnp.maximum(cnt_ref[...], 1.0)


def _pool_call(batch3, dd, z0, z1, y0, y1, b2):
    return pl.pallas_call(
        _pool_body,
        grid=(N // BV,),
        in_specs=[
            pl.BlockSpec((1, 1, BV), lambda i: (i, 0, 0)),
            pl.BlockSpec((BV, 2), lambda i: (i, 0)),
            pl.BlockSpec((BV, DH), lambda i: (i, 0)),
            pl.BlockSpec((BV, DH), lambda i: (i, 0)),
            pl.BlockSpec((BV, DH), lambda i: (i, 0)),
            pl.BlockSpec((BV, DH), lambda i: (i, 0)),
            pl.BlockSpec((1, D), lambda i: (0, 0)),
        ],
        out_specs=pl.BlockSpec((G, D), lambda i: (0, 0)),
        out_shape=jax.ShapeDtypeStruct((G, D), jnp.float32),
        scratch_shapes=[pltpu.VMEM((G, 1), jnp.float32)],
    )(batch3, dd, z0, z1, y0, y1, b2)


def kernel(v, e, batch, W, b):
    er = e.astype(jnp.int32).reshape(2 * E)
    row, col = er[:E], er[E:]
    npd = E_PAD - E
    prow = jnp.arange(npd, dtype=jnp.int32) % N
    pcol = N + (jnp.arange(npd, dtype=jnp.int32) % (NPAD - N))
    rowp = jnp.concatenate([row, prow])
    colp = jnp.concatenate([col, pcol])
    zeros_deg = jnp.zeros((DEG_PT,), jnp.float32)
    ones_ka = jnp.ones((KA,), jnp.float32)
    zeros_rows = jnp.zeros((ROWS_PT, DH), jnp.float32)

    degp = _deg_call()(colp, zeros_deg, ones_ka)
    dd = degp.T

    y0, y1 = _mm_call(dd, v, W)
    z0, z1 = _scatter_call()(rowp, colp, y0, y1, zeros_rows)

    batch3 = batch.astype(jnp.int32).reshape(N // BV, 1, BV)
    b2 = b.reshape(1, D)
    return _pool_call(batch3, dd, z0, z1, y0, y1, b2)


# speedup vs baseline: 1.0357x; 1.0024x over previous
import functools

import jax
import jax.numpy as jnp
from jax import lax
from jax.experimental import pallas as pl
from jax.experimental.pallas import tpu as pltpu
from jax.experimental.pallas import tpu_sc as plsc

N = 10000
E = 160000
D = 256
G = 128
DH = D // 2

NC, NS = 2, 16
NW = NC * NS

KE = 128
E_PAD = 163840
NPAD = 10240
ROWS_PT = NPAD // NS
DEG_PT = 640
DEG_PAD = DEG_PT * NS

KA = E_PAD // NW


def _deg_body(col_hbm, zeros_hbm, ones_hbm, out_hbm, colbuf, onesbuf, acc_sh, sem):
    c = lax.axis_index("c")
    s = lax.axis_index("s")
    wid = c * NS + s
    pltpu.sync_copy(zeros_hbm, acc_sh.at[pl.ds(s * DEG_PT, DEG_PT)])
    pltpu.sync_copy(ones_hbm, onesbuf)
    pltpu.sync_copy(col_hbm.at[pl.ds(wid * KA, KA)], colbuf)
    plsc.subcore_barrier()
    pltpu.sync_copy(onesbuf, acc_sh.at[colbuf], add=True)
    plsc.subcore_barrier()

    @pl.when(c == 0)
    def _():
        pltpu.sync_copy(acc_sh.at[pl.ds(s * DEG_PT, DEG_PT)],
                        out_hbm.at[0, pl.ds(s * DEG_PT, DEG_PT)])

    @pl.when(c == 1)
    def _():
        pltpu.sync_copy(acc_sh.at[pl.ds(s * DEG_PT, DEG_PT)],
                        out_hbm.at[1, pl.ds(s * DEG_PT, DEG_PT)])


@functools.cache
def _deg_call():
    mesh = plsc.VectorSubcoreMesh(core_axis_name="c", subcore_axis_name="s",
                                  num_cores=NC, num_subcores=NS)
    return pl.kernel(
        _deg_body,
        out_type=jax.ShapeDtypeStruct((2, DEG_PAD), jnp.float32),
        mesh=mesh,
        scratch_types=[
            pltpu.VMEM((KA,), jnp.int32),
            pltpu.VMEM((KA,), jnp.float32),
            pltpu.VMEM_SHARED((DEG_PAD,), jnp.float32),
            pltpu.SemaphoreType.DMA,
        ],
    )


NCH = E_PAD // NS // KE
PH = 2
NCHP = NCH // PH


def _copy_idx(src, off, dst):
    for k in range(KE // 16):
        dst[pl.ds(k * 16, 16)] = src[pl.ds(off + k * 16, 16)]


def _scatter_body(row_hbm, col_hbm, y0_hbm, y1_hbm, zrows_hbm, z0_hbm, z1_hbm,
                  rowsb, colsb, colb0, colb1, gbuf0, gbuf1, acc_sh, sem0, sem1):
    c = lax.axis_index("c")
    s = lax.axis_index("s")
    ept = E_PAD // NS
    base = s * ept
    pltpu.sync_copy(zrows_hbm, acc_sh.at[pl.ds(s * ROWS_PT, ROWS_PT), :])
    plsc.subcore_barrier()

    def run(y_hbm):
        def phase(ph, carry0):
            pbase = pl.multiple_of(base + ph * NCHP * KE, 8)
            pltpu.sync_copy(row_hbm.at[pl.ds(pbase, NCHP * KE)], rowsb)
            pltpu.sync_copy(col_hbm.at[pl.ds(pbase, NCHP * KE)], colsb)
            pltpu.async_copy(y_hbm.at[rowsb.at[pl.ds(0, KE)]], gbuf0, sem0)

            def pair(j, carry):
                o0 = pl.multiple_of(2 * j * KE, 128)
                o1 = pl.multiple_of((2 * j + 1) * KE, 128)
                o2 = pl.multiple_of((2 * j + 2) * KE, 128)
                pltpu.async_copy(y_hbm.at[rowsb.at[pl.ds(o1, KE)]], gbuf1, sem1)
                _copy_idx(colsb, o0, colb0)
                pltpu.make_async_copy(y_hbm.at[rowsb.at[pl.ds(o0, KE)]],
                                      gbuf0, sem0).wait()
                pltpu.sync_copy(gbuf0, acc_sh.at[colb0], add=True)

                @pl.when(j < NCHP // 2 - 1)
                def _():
                    pltpu.async_copy(y_hbm.at[rowsb.at[pl.ds(o2, KE)]],
                                     gbuf0, sem0)

                _copy_idx(colsb, o1, colb1)
                pltpu.make_async_copy(y_hbm.at[rowsb.at[pl.ds(o1, KE)]],
                                      gbuf1, sem1).wait()
                pltpu.sync_copy(gbuf1, acc_sh.at[colb1], add=True)
                return carry

            lax.fori_loop(0, NCHP // 2, pair, 0)
            return carry0

        lax.fori_loop(0, PH, phase, 0)

    @pl.when(c == 0)
    def _():
        run(y0_hbm)

    @pl.when(c == 1)
    def _():
        run(y1_hbm)

    plsc.subcore_barrier()

    @pl.when(c == 0)
    def _():
        pltpu.sync_copy(acc_sh.at[pl.ds(s * ROWS_PT, ROWS_PT), :],
                        z0_hbm.at[pl.ds(s * ROWS_PT, ROWS_PT), :])

    @pl.when(c == 1)
    def _():
        pltpu.sync_copy(acc_sh.at[pl.ds(s * ROWS_PT, ROWS_PT), :],
                        z1_hbm.at[pl.ds(s * ROWS_PT, ROWS_PT), :])


@functools.cache
def _scatter_call():
    mesh = plsc.VectorSubcoreMesh(core_axis_name="c", subcore_axis_name="s",
                                  num_cores=NC, num_subcores=NS)
    return pl.kernel(
        _scatter_body,
        out_type=[jax.ShapeDtypeStruct((NPAD, DH), jnp.float32),
                  jax.ShapeDtypeStruct((NPAD, DH), jnp.float32)],
        mesh=mesh,
        scratch_types=[
            pltpu.VMEM((NCHP * KE,), jnp.int32),
            pltpu.VMEM((NCHP * KE,), jnp.int32),
            pltpu.VMEM((KE,), jnp.int32),
            pltpu.VMEM((KE,), jnp.int32),
            pltpu.VMEM((KE, DH), jnp.float32),
            pltpu.VMEM((KE, DH), jnp.float32),
            pltpu.VMEM_SHARED((NPAD, DH), jnp.float32),
            pltpu.SemaphoreType.DMA,
            pltpu.SemaphoreType.DMA,
        ],
    )


BV = 2000


def _mm_body(dd_ref, v_ref, w_ref, y0_ref, y1_ref):
    dis = lax.rsqrt(dd_ref[:, 0:1] + dd_ref[:, 1:2] + 1.0)
    x = jnp.dot(v_ref[...].astype(jnp.bfloat16), w_ref[...].astype(jnp.bfloat16),
                preferred_element_type=jnp.float32)
    y = x * dis
    y0_ref[...] = y[:, :DH]
    y1_ref[...] = y[:, DH:]


def _mm_call(dd, v, W):
    return pl.pallas_call(
        _mm_body,
        grid=(N // BV,),
        in_specs=[
            pl.BlockSpec((BV, 2), lambda i: (i, 0)),
            pl.BlockSpec((BV, D), lambda i: (i, 0)),
            pl.BlockSpec((D, D), lambda i: (0, 0)),
        ],
        out_specs=[
            pl.BlockSpec((BV, DH), lambda i: (i, 0)),
            pl.BlockSpec((BV, DH), lambda i: (i, 0)),
        ],
        out_shape=[jax.ShapeDtypeStruct((N, DH), jnp.float32),
                   jax.ShapeDtypeStruct((N, DH), jnp.float32)],
    )(dd, v, W)


def _pool_body(batch_ref, dd_ref, z0_ref, z1_ref, y0_ref, y1_ref,
               b_ref, out_ref, cnt_ref):
    i = pl.program_id(0)

    @pl.when(i == 0)
    def _():
        out_ref[...] = jnp.zeros_like(out_ref)
        cnt_ref[...] = jnp.zeros_like(cnt_ref)

    dis = lax.rsqrt(dd_ref[:, 0:1] + dd_ref[:, 1:2] + 1.0)
    h0 = jnp.maximum(dis * (z0_ref[...] + y0_ref[...]) + b_ref[:, :DH], 0.0)
    h1 = jnp.maximum(dis * (z1_ref[...] + y1_ref[...]) + b_ref[:, DH:], 0.0)
    bt = batch_ref[0]
    pt = (lax.broadcasted_iota(jnp.int32, (G, BV), 0) == bt).astype(jnp.bfloat16)
    out_ref[:, :DH] += jnp.dot(pt, h0.astype(jnp.bfloat16),
                               preferred_element_type=jnp.float32)
    out_ref[:, DH:] += jnp.dot(pt, h1.astype(jnp.bfloat16),
                               preferred_element_type=jnp.float32)
    cnt_ref[...] += jnp.sum(pt.astype(jnp.float32), axis=1, keepdims=True)

    @pl.when(i == pl.num_programs(0) - 1)
    def _():
        out_ref[...] = out_ref[...] / jnp.maximum(cnt_ref[...], 1.0)


def _pool_call(batch3, dd, z0, z1, y0, y1, b2):
    return pl.pallas_call(
        _pool_body,
        grid=(N // BV,),
        in_specs=[
            pl.BlockSpec((1, 1, BV), lambda i: (i, 0, 0)),
            pl.BlockSpec((BV, 2), lambda i: (i, 0)),
            pl.BlockSpec((BV, DH), lambda i: (i, 0)),
            pl.BlockSpec((BV, DH), lambda i: (i, 0)),
            pl.BlockSpec((BV, DH), lambda i: (i, 0)),
            pl.BlockSpec((BV, DH), lambda i: (i, 0)),
            pl.BlockSpec((1, D), lambda i: (0, 0)),
        ],
        out_specs=pl.BlockSpec((G, D), lambda i: (0, 0)),
        out_shape=jax.ShapeDtypeStruct((G, D), jnp.float32),
        scratch_shapes=[pltpu.VMEM((G, 1), jnp.float32)],
    )(batch3, dd, z0, z1, y0, y1, b2)


def kernel(v, e, batch, W, b):
    er = e.astype(jnp.int32).reshape(2 * E)
    row, col = er[:E], er[E:]
    npd = E_PAD - E
    prow = jnp.arange(npd, dtype=jnp.int32) % N
    pcol = N + (jnp.arange(npd, dtype=jnp.int32) % (NPAD - N))
    rowp = jnp.concatenate([row, prow])
    colp = jnp.concatenate([col, pcol])
    zeros_deg = jnp.zeros((DEG_PT,), jnp.float32)
    ones_ka = jnp.ones((KA,), jnp.float32)
    zeros_rows = jnp.zeros((ROWS_PT, DH), jnp.float32)

    degp = _deg_call()(colp, zeros_deg, ones_ka)
    dd = degp.T

    y0, y1 = _mm_call(dd, v, W)
    z0, z1 = _scatter_call()(rowp, colp, y0, y1, zeros_rows)

    batch3 = batch.astype(jnp.int32).reshape(N // BV, 1, BV)
    b2 = b.reshape(1, D)
    return _pool_call(batch3, dd, z0, z1, y0, y1, b2)
